# baseline (device time: 19800 ns/iter reference)
import jax
import jax.numpy as jnp
from jax import lax
from jax.experimental import pallas as pl
from jax.experimental.pallas import tpu as pltpu

N_DEV = 4
E_LOCAL = 4
HALVES = 2


def kernel(x, router_W, route_idx, expert_W, shared_W):
    n, d = x.shape
    e_total = router_W.shape[1]
    h = expert_W.shape[-1]
    chunk = n // N_DEV
    half = chunk // HALVES
    n_slots = (N_DEV - 1) * HALVES

    def body(x_ref, rw_ref, idx_ref, ew_ref, sw_ref, out_ref,
             part_ref, psc_ref, rs_buf, rsc_buf, ag_buf, agc_buf,
             agsend_ref, agsc_ref, ewb_ref, swb_ref, gate_ref,
             rs_send_sems, rs_recv_sems, rsc_send_sems, rsc_recv_sems,
             ag_send_sems, ag_recv_sems, agc_send_sems, agc_recv_sems):
        my = lax.axis_index("i")

        barrier_sem = pltpu.get_barrier_semaphore()
        for k in range(1, N_DEV):
            pl.semaphore_signal(barrier_sem, inc=1,
                                device_id=(lax.rem(my + k, N_DEV),),
                                device_id_type=pl.DeviceIdType.MESH)
        for le in range(E_LOCAL):
            ewb_ref[pl.ds(le * d, d), :] = ew_ref[le].astype(jnp.bfloat16)
        swb_ref[:, :] = sw_ref[:, :].astype(jnp.bfloat16)

        xv = x_ref[:, :]
        scores = jnp.dot(xv, rw_ref[:, :], preferred_element_type=jnp.float32)
        mx = jnp.max(scores, axis=-1, keepdims=True)
        p = jnp.exp(scores - mx)
        probs = p / jnp.sum(p, axis=-1, keepdims=True)
        oh = lax.broadcasted_iota(jnp.int32, (n, e_total), 1) == idx_ref[:, :]
        gate_ref[:, :] = jnp.sum(jnp.where(oh, probs, 0.0), axis=-1,
                                 keepdims=True)

        pl.semaphore_wait(barrier_sem, N_DEV - 1)

        def compute_part(k):
            row0 = lax.rem(my + k, N_DEV) * chunk
            xk = x_ref[pl.ds(row0, chunk), :]
            idxk = idx_ref[pl.ds(row0, chunk), :]
            xg = xk * gate_ref[pl.ds(row0, chunk), :]
            wx = jnp.concatenate(
                [jnp.where(idxk == my * E_LOCAL + le, xg, 0.0
                           ).astype(jnp.bfloat16) for le in range(E_LOCAL)],
                axis=1)
            return jnp.dot(wx, ewb_ref[:, :],
                           preferred_element_type=jnp.float32)

        def quantize(v):
            rmax = jnp.maximum(jnp.max(jnp.abs(v), axis=-1, keepdims=True),
                               1e-20)
            scale = rmax * (1.0 / 127.0)
            q = jnp.round(v * (127.0 / rmax)).astype(jnp.int8)
            return q, scale

        def remote_copy(src, dst, ssem, rsem, k):
            return pltpu.make_async_remote_copy(
                src_ref=src, dst_ref=dst, send_sem=ssem, recv_sem=rsem,
                device_id=(lax.rem(my + k, N_DEV),),
                device_id_type=pl.DeviceIdType.MESH,
            )

        rs_order = [2, 1, 3]
        rs_rdmas = {}
        for k in rs_order:
            part = compute_part(k)
            for hf in range(HALVES):
                slot = HALVES * (k - 1) + hf
                q, sc = quantize(part[hf * half:(hf + 1) * half, :])
                part_ref[slot] = q
                psc_ref[slot] = sc
                r1 = remote_copy(part_ref.at[slot], rs_buf.at[slot],
                                 rs_send_sems.at[slot],
                                 rs_recv_sems.at[slot], k)
                r1.start()
                r2 = remote_copy(psc_ref.at[slot], rsc_buf.at[slot],
                                 rsc_send_sems.at[slot],
                                 rsc_recv_sems.at[slot], k)
                r2.start()
                rs_rdmas[k, hf] = (r1, r2)

        acc0 = compute_part(0)

        def process_half(hf):
            for k in [1, 3, 2]:
                rs_rdmas[k, hf][0].wait()
                rs_rdmas[k, hf][1].wait()
            red = acc0[hf * half:(hf + 1) * half, :]
            for k in range(1, N_DEV):
                slot = HALVES * (k - 1) + hf
                red = red + rs_buf[slot].astype(jnp.float32) * rsc_buf[slot]
            q, sc = quantize(red)
            agsend_ref[hf] = q
            agsc_ref[hf] = sc
            rdmas = []
            for k in rs_order:
                slot = HALVES * (k - 1) + hf
                r1 = remote_copy(agsend_ref.at[hf], ag_buf.at[slot],
                                 ag_send_sems.at[slot],
                                 ag_recv_sems.at[slot], k)
                r1.start()
                r2 = remote_copy(agsc_ref.at[hf], agc_buf.at[slot],
                                 agc_send_sems.at[slot],
                                 agc_recv_sems.at[slot], k)
                r2.start()
                rdmas.append((k, r1, r2))
            return red, rdmas

        red0, ag0 = process_half(0)
        shared_full = jnp.dot(xv.astype(jnp.bfloat16), swb_ref[:, :],
                              preferred_element_type=jnp.float32)
        red1, ag1 = process_half(1)

        out_ref[:, :] = shared_full
        out_ref[pl.ds(my * chunk, half), :] = (
            out_ref[pl.ds(my * chunk, half), :] + red0)
        out_ref[pl.ds(my * chunk + half, half), :] = (
            out_ref[pl.ds(my * chunk + half, half), :] + red1)

        for hf, rdmas in ((0, ag0), (1, ag1)):
            for k, r1, r2 in rdmas:
                r1.wait()
                r2.wait()
                slot = HALVES * (k - 1) + hf
                row0 = lax.rem(my + N_DEV - k, N_DEV) * chunk + hf * half
                out_ref[pl.ds(row0, half), :] = (
                    out_ref[pl.ds(row0, half), :]
                    + ag_buf[slot].astype(jnp.float32) * agc_buf[slot])

    return pl.pallas_call(
        body,
        out_shape=jax.ShapeDtypeStruct((n, h), jnp.float32),
        in_specs=[pl.BlockSpec(memory_space=pltpu.VMEM)] * 5,
        out_specs=pl.BlockSpec(memory_space=pltpu.VMEM),
        scratch_shapes=[
            pltpu.VMEM((n_slots, half, h), jnp.int8),
            pltpu.VMEM((n_slots, half, 1), jnp.float32),
            pltpu.VMEM((n_slots, half, h), jnp.int8),
            pltpu.VMEM((n_slots, half, 1), jnp.float32),
            pltpu.VMEM((n_slots, half, h), jnp.int8),
            pltpu.VMEM((n_slots, half, 1), jnp.float32),
            pltpu.VMEM((HALVES, half, h), jnp.int8),
            pltpu.VMEM((HALVES, half, 1), jnp.float32),
            pltpu.VMEM((E_LOCAL * d, h), jnp.bfloat16),
            pltpu.VMEM((d, h), jnp.bfloat16),
            pltpu.VMEM((n, 1), jnp.float32),
            pltpu.SemaphoreType.DMA((n_slots,)),
            pltpu.SemaphoreType.DMA((n_slots,)),
            pltpu.SemaphoreType.DMA((n_slots,)),
            pltpu.SemaphoreType.DMA((n_slots,)),
            pltpu.SemaphoreType.DMA((n_slots,)),
            pltpu.SemaphoreType.DMA((n_slots,)),
            pltpu.SemaphoreType.DMA((n_slots,)),
            pltpu.SemaphoreType.DMA((n_slots,)),
        ],
        compiler_params=pltpu.CompilerParams(collective_id=0),
    )(x, router_W, route_idx, expert_W, shared_W)


# device time: 18990 ns/iter; 1.0427x vs baseline; 1.0427x over previous
import jax
import jax.numpy as jnp
from jax import lax
from jax.experimental import pallas as pl
from jax.experimental.pallas import tpu as pltpu

N_DEV = 4
E_LOCAL = 4
HALVES = 2


def kernel(x, router_W, route_idx, expert_W, shared_W):
    n, d = x.shape
    e_total = router_W.shape[1]
    h = expert_W.shape[-1]
    chunk = n // N_DEV
    half = chunk // HALVES
    n_slots = (N_DEV - 1) * HALVES

    def body(x_ref, rw_ref, idx_ref, ew_ref, sw_ref, out_ref,
             part_ref, rs_buf, ag_buf, agsend_ref,
             ewb_ref, swb_ref, gate_ref,
             rs_send_sems, rs_recv_sems, ag_send_sems, ag_recv_sems):
        my = lax.axis_index("i")

        barrier_sem = pltpu.get_barrier_semaphore()
        for k in range(1, N_DEV):
            pl.semaphore_signal(barrier_sem, inc=1,
                                device_id=(lax.rem(my + k, N_DEV),),
                                device_id_type=pl.DeviceIdType.MESH)
        for le in range(E_LOCAL):
            ewb_ref[pl.ds(le * d, d), :] = ew_ref[le].astype(jnp.bfloat16)
        swb_ref[:, :] = sw_ref[:, :].astype(jnp.bfloat16)

        xv = x_ref[:, :]
        scores = jnp.dot(xv, rw_ref[:, :], preferred_element_type=jnp.float32)
        mx = jnp.max(scores, axis=-1, keepdims=True)
        p = jnp.exp(scores - mx)
        probs = p / jnp.sum(p, axis=-1, keepdims=True)
        oh = lax.broadcasted_iota(jnp.int32, (n, e_total), 1) == idx_ref[:, :]
        gate_ref[:, :] = jnp.sum(jnp.where(oh, probs, 0.0), axis=-1,
                                 keepdims=True)

        pl.semaphore_wait(barrier_sem, N_DEV - 1)

        def compute_part(k):
            row0 = lax.rem(my + k, N_DEV) * chunk
            xk = x_ref[pl.ds(row0, chunk), :]
            idxk = idx_ref[pl.ds(row0, chunk), :]
            gk = gate_ref[pl.ds(row0, chunk), :]
            wx = jnp.concatenate(
                [(xk * jnp.where(idxk == my * E_LOCAL + le, gk, 0.0)
                  ).astype(jnp.bfloat16) for le in range(E_LOCAL)],
                axis=1)
            return jnp.dot(wx, ewb_ref[:, :],
                           preferred_element_type=jnp.float32)

        def remote_copy(src, dst, ssem, rsem, k):
            return pltpu.make_async_remote_copy(
                src_ref=src, dst_ref=dst, send_sem=ssem, recv_sem=rsem,
                device_id=(lax.rem(my + k, N_DEV),),
                device_id_type=pl.DeviceIdType.MESH,
            )

        rs_order = [2, 1, 3]
        rs_rdmas = {}
        for k in rs_order:
            part = compute_part(k)
            for hf in range(HALVES):
                slot = HALVES * (k - 1) + hf
                part_ref[slot] = part[hf * half:(hf + 1) * half, :].astype(
                    jnp.bfloat16)
                rdma = remote_copy(part_ref.at[slot], rs_buf.at[slot],
                                   rs_send_sems.at[slot],
                                   rs_recv_sems.at[slot], k)
                rdma.start()
                rs_rdmas[k, hf] = rdma

        acc0 = compute_part(0)

        ag_rdmas = {}
        for hf in range(HALVES):
            for k in [1, 3, 2]:
                rs_rdmas[k, hf].wait()
            red = acc0[hf * half:(hf + 1) * half, :]
            for k in range(1, N_DEV):
                slot = HALVES * (k - 1) + hf
                red = red + rs_buf[slot].astype(jnp.float32)
            agsend_ref[hf] = red.astype(jnp.bfloat16)
            for k in rs_order:
                slot = HALVES * (k - 1) + hf
                rdma = remote_copy(agsend_ref.at[hf], ag_buf.at[slot],
                                   ag_send_sems.at[slot],
                                   ag_recv_sems.at[slot], k)
                rdma.start()
                ag_rdmas[k, hf] = rdma

        out_ref[:, :] = jnp.dot(xv.astype(jnp.bfloat16), swb_ref[:, :],
                                preferred_element_type=jnp.float32)
        for hf in range(HALVES):
            r0 = my * chunk + hf * half
            out_ref[pl.ds(r0, half), :] = (
                out_ref[pl.ds(r0, half), :]
                + agsend_ref[hf].astype(jnp.float32))
        for hf in range(HALVES):
            for k in [1, 3, 2]:
                ag_rdmas[k, hf].wait()
                slot = HALVES * (k - 1) + hf
                row0 = lax.rem(my + N_DEV - k, N_DEV) * chunk + hf * half
                out_ref[pl.ds(row0, half), :] = (
                    out_ref[pl.ds(row0, half), :]
                    + ag_buf[slot].astype(jnp.float32))

    return pl.pallas_call(
        body,
        out_shape=jax.ShapeDtypeStruct((n, h), jnp.float32),
        in_specs=[pl.BlockSpec(memory_space=pltpu.VMEM)] * 5,
        out_specs=pl.BlockSpec(memory_space=pltpu.VMEM),
        scratch_shapes=[
            pltpu.VMEM((n_slots, half, h), jnp.bfloat16),
            pltpu.VMEM((n_slots, half, h), jnp.bfloat16),
            pltpu.VMEM((n_slots, half, h), jnp.bfloat16),
            pltpu.VMEM((HALVES, half, h), jnp.bfloat16),
            pltpu.VMEM((E_LOCAL * d, h), jnp.bfloat16),
            pltpu.VMEM((d, h), jnp.bfloat16),
            pltpu.VMEM((n, 1), jnp.float32),
            pltpu.SemaphoreType.DMA((n_slots,)),
            pltpu.SemaphoreType.DMA((n_slots,)),
            pltpu.SemaphoreType.DMA((n_slots,)),
            pltpu.SemaphoreType.DMA((n_slots,)),
        ],
        compiler_params=pltpu.CompilerParams(collective_id=0),
    )(x, router_W, route_idx, expert_W, shared_W)
